# baseline (speedup 1.0000x reference)
import functools

import jax
import jax.numpy as jnp
from jax import lax
from jax.experimental import pallas as pl
from jax.experimental.pallas import tpu as pltpu
from jax.experimental.pallas import tpu_sc as plsc

BATCH = 4096
SEQ = 200
N = BATCH * SEQ
D_ITEM = 64
D_FAKE = 16
D_OUT = D_ITEM + D_FAKE
NUM_WORKERS = 32

NUM_SLICES = 4
B_SLICE = BATCH // NUM_SLICES
N_SLICE = B_SLICE * SEQ
B_PER_W = B_SLICE // NUM_WORKERS

TC_TILE_B = 16
TC_ROWS = TC_TILE_B * SEQ
TC_GRID = N_SLICE // TC_ROWS


def _sc_gather(ids, item_table):
    mesh = plsc.VectorSubcoreMesh(core_axis_name="c", subcore_axis_name="s")

    @functools.partial(
        pl.kernel,
        mesh=mesh,
        compiler_params=pltpu.CompilerParams(use_tc_tiling_on_sc=False),
        out_type=jax.ShapeDtypeStruct((N_SLICE, D_ITEM), jnp.float32),
        scratch_types=[
            pltpu.VMEM((B_PER_W, SEQ), jnp.int32),
            pltpu.VMEM((SEQ, D_ITEM), jnp.float32),
            pltpu.VMEM((SEQ, D_ITEM), jnp.float32),
            pltpu.SemaphoreType.DMA,
            pltpu.SemaphoreType.DMA,
            pltpu.SemaphoreType.DMA,
            pltpu.SemaphoreType.DMA,
        ],
    )
    def k(ids_hbm, item_hbm, out_hbm, ids_v, st0, st1, gs0, gs1, ws0, ws1):
        wid = lax.axis_index("s") * 2 + lax.axis_index("c")
        b0 = wid * B_PER_W
        pltpu.sync_copy(ids_hbm.at[pl.ds(b0, B_PER_W)], ids_v)
        stage = (st0, st1)
        gsems = (gs0, gs1)
        wsems = (ws0, ws1)

        def issue_gathers(c, par, sem):
            return [
                pltpu.async_copy(
                    item_hbm.at[ids_v.at[c, pl.ds(k * 40, 40)]],
                    stage[par].at[pl.ds(k * 40, 40)], sem)
                for k in range(5)
            ]

        def issue_writeback(c, par, sem):
            return pltpu.async_copy(
                stage[par],
                out_hbm.at[pl.ds((b0 + c) * SEQ, SEQ)], sem)

        for cp in issue_gathers(0, 0, gsems[0]):
            cp.wait()

        def pair_body(p, carry):
            for par in range(2):
                c = p * 2 + par
                npar = 1 - par
                g = lax.rem(c + 1, B_PER_W)
                gcps = issue_gathers(g, npar, gsems[npar])
                wcp = issue_writeback(c, par, wsems[par])
                for cp in gcps:
                    cp.wait()
                wcp.wait()
            return carry

        lax.fori_loop(0, B_PER_W // 2, pair_body, 0)

    return k(ids, item_table)


def _tc_assemble(gathered, pos_tiled, fids3, fake_table, prev, slice_idx):
    def body(*refs):
        gath_hbm, pos_ref, fid_ref, fake_ref = refs[:4]
        out_ref = refs[-3]
        gbuf, sem = refs[-2], refs[-1]
        g = pl.program_id(0)
        pltpu.make_async_copy(
            gath_hbm.at[pl.ds(g * TC_ROWS, TC_ROWS)], gbuf, sem).start()
        f = fid_ref[0, 0, :].reshape(TC_ROWS, 1)
        fb = fake_ref[...]
        r0 = fb[0:1, :]
        r1 = fb[1:2, :]
        r2 = fb[2:3, :]
        fe = jnp.where(f == 0, r0, jnp.where(f == 1, r1, r2))
        out_ref[:, D_ITEM:D_OUT] = fe
        pltpu.make_async_copy(
            gath_hbm.at[pl.ds(g * TC_ROWS, TC_ROWS)], gbuf, sem).wait()
        out_ref[:, 0:D_ITEM] = gbuf[...] + pos_ref[...]

    in_specs = [
        pl.BlockSpec(memory_space=pl.ANY),
        pl.BlockSpec((TC_ROWS, D_ITEM), lambda g: (0, 0)),
        pl.BlockSpec((1, 1, TC_ROWS), lambda g: (g, 0, 0)),
        pl.BlockSpec((3, D_FAKE), lambda g: (0, 0)),
    ]
    args = [gathered, pos_tiled, fids3, fake_table]
    aliases = {}
    if prev is not None:
        in_specs.append(pl.BlockSpec(memory_space=pl.ANY))
        args.append(prev)
        aliases = {4: 0}
    return pl.pallas_call(
        body,
        grid=(TC_GRID,),
        in_specs=in_specs,
        out_specs=pl.BlockSpec(
            (TC_ROWS, D_OUT),
            lambda g, _si=slice_idx: (_si * TC_GRID + g, 0)),
        out_shape=jax.ShapeDtypeStruct((N, D_OUT), jnp.float32),
        input_output_aliases=aliases,
        scratch_shapes=[
            pltpu.VMEM((TC_ROWS, D_ITEM), jnp.float32),
            pltpu.SemaphoreType.DMA,
        ],
    )(*args)


def kernel(input_ids, fake_ids, item_table, pos_table, fake_table):
    ids = input_ids.astype(jnp.int32)
    fids = fake_ids.astype(jnp.int32)
    pos_tiled = jnp.tile(pos_table, (TC_TILE_B, 1))

    gathered = [
        _sc_gather(ids[si * B_SLICE:(si + 1) * B_SLICE], item_table)
        for si in range(NUM_SLICES)
    ]
    out = None
    for si in range(NUM_SLICES):
        fids3 = fids[si * B_SLICE:(si + 1) * B_SLICE].reshape(
            TC_GRID, 1, TC_ROWS)
        out = _tc_assemble(gathered[si], pos_tiled, fids3, fake_table,
                           out, si)
    return out.reshape(BATCH, SEQ, D_OUT)


# speedup vs baseline: 1.1097x; 1.1097x over previous
import functools

import jax
import jax.numpy as jnp
from jax import lax
from jax.experimental import pallas as pl
from jax.experimental.pallas import tpu as pltpu
from jax.experimental.pallas import tpu_sc as plsc

BATCH = 4096
SEQ = 200
N = BATCH * SEQ
D_ITEM = 64
D_FAKE = 16
D_OUT = D_ITEM + D_FAKE
NUM_WORKERS = 32

NUM_SLICES = 4
B_SLICE = BATCH // NUM_SLICES
N_SLICE = B_SLICE * SEQ
B_PER_W = B_SLICE // NUM_WORKERS

TC_TILE_B = 16
TC_ROWS = TC_TILE_B * SEQ
TC_GRID = N_SLICE // TC_ROWS


def _sc_gather(ids, item_table):
    mesh = plsc.VectorSubcoreMesh(core_axis_name="c", subcore_axis_name="s")

    @functools.partial(
        pl.kernel,
        mesh=mesh,
        compiler_params=pltpu.CompilerParams(use_tc_tiling_on_sc=False),
        out_type=jax.ShapeDtypeStruct((N_SLICE, D_ITEM), jnp.float32),
        scratch_types=[
            pltpu.VMEM((B_PER_W, SEQ), jnp.int32),
            pltpu.VMEM((SEQ, D_ITEM), jnp.float32),
            pltpu.VMEM((SEQ, D_ITEM), jnp.float32),
            pltpu.SemaphoreType.DMA,
            pltpu.SemaphoreType.DMA,
            pltpu.SemaphoreType.DMA,
            pltpu.SemaphoreType.DMA,
        ],
    )
    def k(ids_hbm, item_hbm, out_hbm, ids_v, st0, st1, gs0, gs1, ws0, ws1):
        wid = lax.axis_index("s") * 2 + lax.axis_index("c")
        b0 = wid * B_PER_W
        pltpu.sync_copy(ids_hbm.at[pl.ds(b0, B_PER_W)], ids_v)
        stage = (st0, st1)
        gsems = (gs0, gs1)
        wsems = (ws0, ws1)

        def issue_gathers(c, par, sem):
            return [
                pltpu.async_copy(
                    item_hbm.at[ids_v.at[c, pl.ds(k * 40, 40)]],
                    stage[par].at[pl.ds(k * 40, 40)], sem)
                for k in range(5)
            ]

        def issue_writeback(c, par, sem):
            return pltpu.async_copy(
                stage[par],
                out_hbm.at[pl.ds((b0 + c) * SEQ, SEQ)], sem)

        for cp in issue_gathers(0, 0, gsems[0]):
            cp.wait()

        def pair_body(p, carry):
            for par in range(2):
                c = p * 2 + par
                npar = 1 - par
                g = lax.rem(c + 1, B_PER_W)
                gcps = issue_gathers(g, npar, gsems[npar])
                wcp = issue_writeback(c, par, wsems[par])
                for cp in gcps:
                    cp.wait()
                wcp.wait()
            return carry

        lax.fori_loop(0, B_PER_W // 2, pair_body, 0)

    return k(ids, item_table)


def _tc_assemble(gathered, pos_tiled, fids3, fake_table, prev, slice_idx):
    def body(*refs):
        gath_ref, pos_ref, fid_ref, fake_ref = refs[:4]
        out_ref = refs[-1]
        out_ref[:, 0:D_ITEM] = gath_ref[...] + pos_ref[...]
        f = fid_ref[0, 0, :].reshape(TC_ROWS, 1)
        fb = fake_ref[...]
        r0 = fb[0:1, :]
        r1 = fb[1:2, :]
        r2 = fb[2:3, :]
        fe = jnp.where(f == 0, r0, jnp.where(f == 1, r1, r2))
        out_ref[:, D_ITEM:D_OUT] = fe

    in_specs = [
        pl.BlockSpec((TC_ROWS, D_ITEM), lambda g: (g, 0)),
        pl.BlockSpec((TC_ROWS, D_ITEM), lambda g: (0, 0)),
        pl.BlockSpec((1, 1, TC_ROWS), lambda g: (g, 0, 0)),
        pl.BlockSpec((3, D_FAKE), lambda g: (0, 0)),
    ]
    args = [gathered, pos_tiled, fids3, fake_table]
    aliases = {}
    if prev is not None:
        in_specs.append(pl.BlockSpec(memory_space=pl.ANY))
        args.append(prev)
        aliases = {4: 0}
    return pl.pallas_call(
        body,
        grid=(TC_GRID,),
        in_specs=in_specs,
        out_specs=pl.BlockSpec(
            (TC_ROWS, D_OUT),
            lambda g, _si=slice_idx: (_si * TC_GRID + g, 0)),
        out_shape=jax.ShapeDtypeStruct((N, D_OUT), jnp.float32),
        input_output_aliases=aliases,
    )(*args)


def kernel(input_ids, fake_ids, item_table, pos_table, fake_table):
    ids = input_ids.astype(jnp.int32)
    fids = fake_ids.astype(jnp.int32)
    pos_tiled = jnp.tile(pos_table, (TC_TILE_B, 1))

    gathered = [
        _sc_gather(ids[si * B_SLICE:(si + 1) * B_SLICE], item_table)
        for si in range(NUM_SLICES)
    ]
    out = None
    for si in range(NUM_SLICES):
        fids3 = fids[si * B_SLICE:(si + 1) * B_SLICE].reshape(
            TC_GRID, 1, TC_ROWS)
        out = _tc_assemble(gathered[si], pos_tiled, fids3, fake_table,
                           out, si)
    return out.reshape(BATCH, SEQ, D_OUT)


# speedup vs baseline: 1.1738x; 1.0578x over previous
import functools

import jax
import jax.numpy as jnp
from jax import lax
from jax.experimental import pallas as pl
from jax.experimental.pallas import tpu as pltpu
from jax.experimental.pallas import tpu_sc as plsc

BATCH = 4096
SEQ = 200
N = BATCH * SEQ
D_ITEM = 64
D_FAKE = 16
D_OUT = D_ITEM + D_FAKE
NUM_WORKERS = 32

NUM_SLICES = 4
B_SLICE = BATCH // NUM_SLICES
N_SLICE = B_SLICE * SEQ
B_PER_W = B_SLICE // NUM_WORKERS

TC_TILE_B = 64
TC_ROWS = TC_TILE_B * SEQ
TC_GRID = N_SLICE // TC_ROWS


def _sc_gather(ids, item_table):
    mesh = plsc.VectorSubcoreMesh(core_axis_name="c", subcore_axis_name="s")

    @functools.partial(
        pl.kernel,
        mesh=mesh,
        compiler_params=pltpu.CompilerParams(use_tc_tiling_on_sc=False),
        out_type=jax.ShapeDtypeStruct((N_SLICE, D_ITEM), jnp.float32),
        scratch_types=[
            pltpu.VMEM((B_PER_W, SEQ), jnp.int32),
            pltpu.VMEM((SEQ, D_ITEM), jnp.float32),
            pltpu.VMEM((SEQ, D_ITEM), jnp.float32),
            pltpu.SemaphoreType.DMA,
            pltpu.SemaphoreType.DMA,
            pltpu.SemaphoreType.DMA,
            pltpu.SemaphoreType.DMA,
        ],
    )
    def k(ids_hbm, item_hbm, out_hbm, ids_v, st0, st1, gs0, gs1, ws0, ws1):
        wid = lax.axis_index("s") * 2 + lax.axis_index("c")
        b0 = wid * B_PER_W
        pltpu.sync_copy(ids_hbm.at[pl.ds(b0, B_PER_W)], ids_v)
        stage = (st0, st1)
        gsems = (gs0, gs1)
        wsems = (ws0, ws1)

        def issue_gathers(c, par, sem):
            return [
                pltpu.async_copy(
                    item_hbm.at[ids_v.at[c, pl.ds(k * 40, 40)]],
                    stage[par].at[pl.ds(k * 40, 40)], sem)
                for k in range(5)
            ]

        def issue_writeback(c, par, sem):
            return pltpu.async_copy(
                stage[par],
                out_hbm.at[pl.ds((b0 + c) * SEQ, SEQ)], sem)

        for cp in issue_gathers(0, 0, gsems[0]):
            cp.wait()

        def pair_body(p, carry):
            for par in range(2):
                c = p * 2 + par
                npar = 1 - par
                g = lax.rem(c + 1, B_PER_W)
                gcps = issue_gathers(g, npar, gsems[npar])
                wcp = issue_writeback(c, par, wsems[par])
                for cp in gcps:
                    cp.wait()
                wcp.wait()
            return carry

        lax.fori_loop(0, B_PER_W // 2, pair_body, 0)

    return k(ids, item_table)


def _tc_assemble(gathered, pos_tiled, fids3, fake_table, prev, slice_idx):
    def body(*refs):
        gath_ref, pos_ref, fid_ref, fake_ref = refs[:4]
        out_ref = refs[-1]
        out_ref[:, 0:D_ITEM] = gath_ref[...] + pos_ref[...]
        f = fid_ref[0, 0, :].reshape(TC_ROWS, 1)
        fb = fake_ref[...]
        r0 = fb[0:1, :]
        r1 = fb[1:2, :]
        r2 = fb[2:3, :]
        fe = jnp.where(f == 0, r0, jnp.where(f == 1, r1, r2))
        out_ref[:, D_ITEM:D_OUT] = fe

    in_specs = [
        pl.BlockSpec((TC_ROWS, D_ITEM), lambda g: (g, 0)),
        pl.BlockSpec((TC_ROWS, D_ITEM), lambda g: (0, 0)),
        pl.BlockSpec((1, 1, TC_ROWS), lambda g: (g, 0, 0)),
        pl.BlockSpec((3, D_FAKE), lambda g: (0, 0)),
    ]
    args = [gathered, pos_tiled, fids3, fake_table]
    aliases = {}
    if prev is not None:
        in_specs.append(pl.BlockSpec(memory_space=pl.ANY))
        args.append(prev)
        aliases = {4: 0}
    return pl.pallas_call(
        body,
        grid=(TC_GRID,),
        in_specs=in_specs,
        out_specs=pl.BlockSpec(
            (TC_ROWS, D_OUT),
            lambda g, _si=slice_idx: (_si * TC_GRID + g, 0)),
        out_shape=jax.ShapeDtypeStruct((N, D_OUT), jnp.float32),
        input_output_aliases=aliases,
    )(*args)


def kernel(input_ids, fake_ids, item_table, pos_table, fake_table):
    ids = input_ids.astype(jnp.int32)
    fids = fake_ids.astype(jnp.int32)
    pos_tiled = jnp.tile(pos_table, (TC_TILE_B, 1))

    gathered = [
        _sc_gather(ids[si * B_SLICE:(si + 1) * B_SLICE], item_table)
        for si in range(NUM_SLICES)
    ]
    out = None
    for si in range(NUM_SLICES):
        fids3 = fids[si * B_SLICE:(si + 1) * B_SLICE].reshape(
            TC_GRID, 1, TC_ROWS)
        out = _tc_assemble(gathered[si], pos_tiled, fids3, fake_table,
                           out, si)
    return out.reshape(BATCH, SEQ, D_OUT)
